# baseline (speedup 1.0000x reference)
import jax
import jax.numpy as jnp
from jax.experimental import pallas as pl
from jax.experimental.pallas import tpu as pltpu

_N = 54
_E = 2862
_D_IN, _D_HID, _D_OUT, _D_GLOB = 512, 512, 256, 64
_EMB = _N * _D_OUT
_WO1R = _EMB + _D_GLOB


def _net_kernel(ei_ref, feat_ref, glob_ref,
                W1_hbm, b1_ref, W2_hbm, b2_ref, W3_hbm, b3_ref,
                Wg1T_ref, bg1_ref, Wg2_ref, bg2_ref, Wg3_ref, bg3_ref,
                Wo1T_hbm, bo1_ref, Wo2T_ref, bo2_ref, out_ref,
                w1_s, w2_s, w3_s, wo1t_s,
                s1, s1b, s2, s2b, s3, s4, s4b):
    f32 = jnp.float32

    def split_copy(hbm, vmem_s, sem_a, sem_b, rows):
        half = (rows // 2) // 8 * 8
        ca = pltpu.make_async_copy(hbm.at[pl.ds(0, half), :],
                                   vmem_s.at[pl.ds(0, half), :], sem_a)
        cb = pltpu.make_async_copy(hbm.at[pl.ds(half, rows - half), :],
                                   vmem_s.at[pl.ds(half, rows - half), :],
                                   sem_b)
        ca.start()
        cb.start()
        return ca, cb

    cp1a, cp1b = split_copy(W1_hbm, w1_s, s1, s1b, _D_IN)
    cp2a, cp2b = split_copy(W2_hbm, w2_s, s2, s2b, _D_HID)
    cp3 = pltpu.make_async_copy(W3_hbm, w3_s, s3)
    cp3.start()
    cp4a, cp4b = split_copy(Wo1T_hbm, wo1t_s, s4, s4b, 85)

    src = ei_ref[0:1, :]
    dst = ei_ref[1:2, :]
    node_iota = jax.lax.broadcasted_iota(jnp.int32, (_N, _E), 0)
    oh_src = (src == node_iota).astype(f32)
    oh_dst = (dst == node_iota).astype(f32)
    A = jax.lax.dot_general(oh_dst, oh_src, (((1,), (1,)), ((), ())),
                            preferred_element_type=f32)
    deg_out = jnp.sum(A, axis=0, keepdims=True)
    deg_in = jnp.sum(A, axis=1, keepdims=True)
    n_out = jax.lax.rsqrt(jnp.maximum(deg_out, 1.0))
    n_in = jax.lax.rsqrt(jnp.maximum(deg_in, 1.0))
    Ahat = A * n_in * n_out

    g = glob_ref[...].reshape(1, _D_GLOB)
    g = jnp.maximum(
        jax.lax.dot_general(g, Wg1T_ref[...], (((1,), (1,)), ((), ())),
                            preferred_element_type=f32) + bg1_ref[...], 0.0)
    g = jnp.maximum(jnp.dot(g, Wg2_ref[...]) + bg2_ref[...], 0.0)
    g = jnp.maximum(jnp.dot(g, Wg3_ref[...]) + bg3_ref[...], 0.0)

    ax = jnp.dot(Ahat, feat_ref[...], preferred_element_type=f32)
    cp1a.wait()
    cp1b.wait()
    h = jnp.maximum(jnp.dot(ax, w1_s[...], preferred_element_type=f32)
                    + b1_ref[...], 0.0)
    ah = jnp.dot(Ahat, h, preferred_element_type=f32)
    cp2a.wait()
    cp2b.wait()
    h = jnp.maximum(jnp.dot(ah, w2_s[...], preferred_element_type=f32)
                    + b2_ref[...], 0.0)
    ah = jnp.dot(Ahat, h, preferred_element_type=f32)
    cp3.wait()
    emb = jnp.maximum(jnp.dot(ah, w3_s[...], preferred_element_type=f32)
                      + b3_ref[...], 0.0)

    cat = jnp.concatenate([emb.reshape(1, _EMB), g], axis=1)
    cp4a.wait()
    cp4b.wait()
    out1 = (jax.lax.dot_general(cat, wo1t_s[...], (((1,), (1,)), ((), ())),
                                preferred_element_type=f32)
            + bo1_ref[...])
    out1 = jnp.maximum(out1, 0.0)
    out2 = (jnp.sum(out1 * Wo2T_ref[...], axis=1, keepdims=True)
            + bo2_ref[...])
    out_ref[...] = jax.nn.sigmoid(out2)


def kernel(feat, edge_index, globalFeats, isTrain,
           W1, b1, W2, b2, W3, b3,
           Wg1, bg1, Wg2, bg2, Wg3, bg3,
           Wo1, bo1, Wo2, bo2):
    f32 = jnp.float32
    vmem = pl.BlockSpec(memory_space=pltpu.MemorySpace.VMEM)
    hbm = pl.BlockSpec(memory_space=pltpu.MemorySpace.HBM)
    out = pl.pallas_call(
        _net_kernel,
        out_shape=jax.ShapeDtypeStruct((1, 1), f32),
        in_specs=[vmem, vmem, vmem,
                  hbm, vmem, hbm, vmem, hbm, vmem,
                  vmem, vmem, vmem, vmem, vmem, vmem,
                  hbm, vmem, vmem, vmem],
        out_specs=vmem,
        scratch_shapes=[
            pltpu.VMEM((_D_IN, _D_HID), f32),
            pltpu.VMEM((_D_HID, _D_HID), f32),
            pltpu.VMEM((_D_HID, _D_OUT), f32),
            pltpu.VMEM((85, _WO1R), f32),
            pltpu.SemaphoreType.DMA,
            pltpu.SemaphoreType.DMA,
            pltpu.SemaphoreType.DMA,
            pltpu.SemaphoreType.DMA,
            pltpu.SemaphoreType.DMA,
            pltpu.SemaphoreType.DMA,
            pltpu.SemaphoreType.DMA,
        ],
    )(edge_index.astype(jnp.int32), feat, globalFeats,
      W1, b1, W2, b2, W3, b3,
      Wg1.T, bg1, Wg2, bg2, Wg3, bg3,
      Wo1.T, bo1, Wo2.T, bo2)
    return out.reshape(1)


# speedup vs baseline: 1.0226x; 1.0226x over previous
import jax
import jax.numpy as jnp
from jax.experimental import pallas as pl
from jax.experimental.pallas import tpu as pltpu

_N = 54
_E = 2862
_D_IN, _D_HID, _D_OUT, _D_GLOB = 512, 512, 256, 64
_EMB = _N * _D_OUT
_WO1R = _EMB + _D_GLOB


def _net_kernel(ei_ref, feat_ref, glob_ref,
                W1_hbm, b1_ref, W2_hbm, b2_ref, W3_hbm, b3_ref,
                Wg1T_ref, bg1_ref, Wg2_ref, bg2_ref, Wg3_ref, bg3_ref,
                Wo1T_hbm, bo1_ref, Wo2T_ref, bo2_ref, out_ref,
                w1_s, w2_s, w3_s, wo1t_s, s1, s2, s3, s4):
    f32 = jnp.float32
    cp1 = pltpu.make_async_copy(W1_hbm, w1_s, s1)
    cp1.start()
    cp2 = pltpu.make_async_copy(W2_hbm, w2_s, s2)
    cp2.start()
    cp3 = pltpu.make_async_copy(W3_hbm, w3_s, s3)
    cp3.start()
    cp4 = pltpu.make_async_copy(Wo1T_hbm, wo1t_s, s4)
    cp4.start()

    src = ei_ref[0:1, :]
    dst = ei_ref[1:2, :]
    node_iota = jax.lax.broadcasted_iota(jnp.int32, (_N, _E), 0)
    oh_src = (src == node_iota).astype(f32)
    oh_dst = (dst == node_iota).astype(f32)
    A = jax.lax.dot_general(oh_dst, oh_src, (((1,), (1,)), ((), ())),
                            preferred_element_type=f32)
    deg_out = jnp.sum(A, axis=0, keepdims=True)
    deg_in = jnp.sum(A, axis=1, keepdims=True)
    n_out = jax.lax.rsqrt(jnp.maximum(deg_out, 1.0))
    n_in = jax.lax.rsqrt(jnp.maximum(deg_in, 1.0))
    Ahat = A * n_in * n_out

    g = glob_ref[...].reshape(1, _D_GLOB)
    g = jnp.maximum(
        jax.lax.dot_general(g, Wg1T_ref[...], (((1,), (1,)), ((), ())),
                            preferred_element_type=f32) + bg1_ref[...], 0.0)
    g = jnp.maximum(jnp.dot(g, Wg2_ref[...]) + bg2_ref[...], 0.0)
    g = jnp.maximum(jnp.dot(g, Wg3_ref[...]) + bg3_ref[...], 0.0)

    ax = jnp.dot(Ahat, feat_ref[...], preferred_element_type=f32)
    cp1.wait()
    h = jnp.maximum(jnp.dot(ax, w1_s[...], preferred_element_type=f32)
                    + b1_ref[...], 0.0)
    ah = jnp.dot(Ahat, h, preferred_element_type=f32)
    cp2.wait()
    h = jnp.maximum(jnp.dot(ah, w2_s[...], preferred_element_type=f32)
                    + b2_ref[...], 0.0)
    ah = jnp.dot(Ahat, h, preferred_element_type=f32)
    cp3.wait()
    emb = jnp.maximum(jnp.dot(ah, w3_s[...], preferred_element_type=f32)
                      + b3_ref[...], 0.0)

    cat = jnp.concatenate([emb.reshape(1, _EMB), g], axis=1)
    cp4.wait()
    out1 = (jax.lax.dot_general(cat, wo1t_s[...], (((1,), (1,)), ((), ())),
                                preferred_element_type=f32)
            + bo1_ref[...])
    out1 = jnp.maximum(out1, 0.0)
    out2 = (jnp.sum(out1 * Wo2T_ref[...], axis=1, keepdims=True)
            + bo2_ref[...])
    out_ref[...] = jax.nn.sigmoid(out2)


def kernel(feat, edge_index, globalFeats, isTrain,
           W1, b1, W2, b2, W3, b3,
           Wg1, bg1, Wg2, bg2, Wg3, bg3,
           Wo1, bo1, Wo2, bo2):
    f32 = jnp.float32
    vmem = pl.BlockSpec(memory_space=pltpu.MemorySpace.VMEM)
    hbm = pl.BlockSpec(memory_space=pltpu.MemorySpace.HBM)
    out = pl.pallas_call(
        _net_kernel,
        out_shape=jax.ShapeDtypeStruct((1, 1), f32),
        in_specs=[vmem, vmem, vmem,
                  hbm, vmem, hbm, vmem, hbm, vmem,
                  vmem, vmem, vmem, vmem, vmem, vmem,
                  hbm, vmem, vmem, vmem],
        out_specs=vmem,
        scratch_shapes=[
            pltpu.VMEM((_D_IN, _D_HID), f32),
            pltpu.VMEM((_D_HID, _D_HID), f32),
            pltpu.VMEM((_D_HID, _D_OUT), f32),
            pltpu.VMEM((85, _WO1R), f32),
            pltpu.SemaphoreType.DMA,
            pltpu.SemaphoreType.DMA,
            pltpu.SemaphoreType.DMA,
            pltpu.SemaphoreType.DMA,
        ],
    )(edge_index.astype(jnp.int32), feat, globalFeats,
      W1, b1, W2, b2, W3, b3,
      Wg1.T, bg1, Wg2, bg2, Wg3, bg3,
      Wo1.T, bo1, Wo2.T, bo2)
    return out.reshape(1)
